# Initial kernel scaffold; baseline (speedup 1.0000x reference)
import jax
import jax.numpy as jnp
from jax.experimental import pallas as pl


def kernel(metadata, image, params):
    raise NotImplementedError("write your pallas kernel here")


# speedup vs baseline: 1.2851x; 1.2851x over previous
import functools
import math

import jax
import jax.numpy as jnp
from jax.experimental import pallas as pl
from jax.experimental.pallas import tpu as pltpu

_SQRT2 = math.sqrt(2.0)


def _gelu(x):
    return 0.5 * x * (1.0 + jax.lax.erf(x / _SQRT2))


def _ln(x, g, b, eps):
    m = jnp.mean(x, axis=-1, keepdims=True)
    v = jnp.mean((x - m) ** 2, axis=-1, keepdims=True)
    return (x - m) * jax.lax.rsqrt(v + eps) * g + b


def _full_spec(shape):
    return pl.BlockSpec(shape, lambda i: (0,) * len(shape))


def _batch_spec(shape, tb):
    bs = (tb,) + tuple(shape[1:])
    return pl.BlockSpec(bs, lambda i: (i,) + (0,) * (len(shape) - 1))


def _stem_body(x_ref, w_ref, b_ref, lg_ref, lb_ref, o_ref):
    tb, p, k = x_ref.shape
    y = jnp.dot(x_ref[...].reshape(tb * p, k), w_ref[...],
                preferred_element_type=jnp.float32) + b_ref[...]
    y = _ln(y, lg_ref[...], lb_ref[...], 1e-6)
    o_ref[...] = y.reshape(tb, p, -1)


def _stem_call(xp, w, b, lg, lb):
    B, P, K = xp.shape
    C = w.shape[1]
    TB = 32
    return pl.pallas_call(
        _stem_body,
        grid=(B // TB,),
        in_specs=[
            _batch_spec(xp.shape, TB),
            _full_spec(w.shape), _full_spec(b.shape),
            _full_spec(lg.shape), _full_spec(lb.shape),
        ],
        out_specs=_batch_spec((B, P, C), TB),
        out_shape=jax.ShapeDtypeStruct((B, P, C), jnp.float32),
    )(xp, w, b, lg, lb)


def _block_hw_body(x_ref, dwk_ref, dwb_ref, lg_ref, lb_ref, w1_ref, b1_ref,
                   w2_ref, b2_ref, g_ref, o_ref, pad_ref):
    tb, H, W, C = x_ref.shape
    x = x_ref[...]
    pad_ref[...] = jnp.zeros_like(pad_ref)
    pad_ref[:, 3:H + 3, 3:W + 3, :] = x
    y = jnp.zeros((tb, H, W, C), jnp.float32)
    for k in range(49):
        dh, dw = k // 7, k % 7
        y = y + pad_ref[:, dh:dh + H, dw:dw + W, :] * dwk_ref[k]
    y = y + dwb_ref[...]
    y = _ln(y, lg_ref[...], lb_ref[...], 1e-6)
    h = jnp.dot(y.reshape(tb * H * W, C), w1_ref[...],
                preferred_element_type=jnp.float32) + b1_ref[...]
    h = _gelu(h)
    o = jnp.dot(h, w2_ref[...], preferred_element_type=jnp.float32) + b2_ref[...]
    o_ref[...] = x + g_ref[...] * o.reshape(tb, H, W, C)


def _block_hw_call(x, p, tb):
    B, H, W, C = x.shape
    dwk = p['dw'].reshape(49, C)
    args = (x, dwk, p['dwb'].reshape(1, C), p['lg'].reshape(1, C),
            p['lb'].reshape(1, C), p['w1'], p['b1'].reshape(1, -1),
            p['w2'], p['b2'].reshape(1, C), p['g'].reshape(1, C))
    return pl.pallas_call(
        _block_hw_body,
        grid=(B // tb,),
        in_specs=[_batch_spec(x.shape, tb)] + [_full_spec(a.shape) for a in args[1:]],
        out_specs=_batch_spec(x.shape, tb),
        out_shape=jax.ShapeDtypeStruct(x.shape, jnp.float32),
        scratch_shapes=[pltpu.VMEM((tb, H + 6, W + 6, C), jnp.float32)],
    )(*args)


def _block_flat_body(x_ref, kT_ref, dwb_ref, lg_ref, lb_ref, w1_ref, b1_ref,
                     w2_ref, b2_ref, g_ref, o_ref):
    tb, P, C = x_ref.shape
    x = x_ref[...]
    y = jnp.zeros((tb, P, C), jnp.float32)
    for q in range(P):
        y = y + x[:, q:q + 1, :] * kT_ref[q][None]
    y = y + dwb_ref[...]
    y = _ln(y, lg_ref[...], lb_ref[...], 1e-6)
    h = jnp.dot(y.reshape(tb * P, C), w1_ref[...],
                preferred_element_type=jnp.float32) + b1_ref[...]
    h = _gelu(h)
    o = jnp.dot(h, w2_ref[...], preferred_element_type=jnp.float32) + b2_ref[...]
    o_ref[...] = x + g_ref[...] * o.reshape(tb, P, C)


def _mix_matrix(dw, hw):
    C = dw.shape[-1]
    k2 = dw.reshape(7, 7, C)
    rows = []
    for q in range(hw * hw):
        hq, wq = q // hw, q % hw
        row = []
        for p_ in range(hw * hw):
            hp, wp = p_ // hw, p_ % hw
            row.append(k2[hq - hp + 3, wq - wp + 3])
        rows.append(jnp.stack(row))
    return jnp.stack(rows)


def _block_flat_call(x, p, hw, tb):
    B, P, C = x.shape
    kT = _mix_matrix(p['dw'], hw)
    args = (x, kT, p['dwb'].reshape(1, 1, C), p['lg'].reshape(1, 1, C),
            p['lb'].reshape(1, 1, C), p['w1'], p['b1'].reshape(1, -1),
            p['w2'], p['b2'].reshape(1, C), p['g'].reshape(1, 1, C))
    return pl.pallas_call(
        _block_flat_body,
        grid=(B // tb,),
        in_specs=[_batch_spec(x.shape, tb)] + [_full_spec(a.shape) for a in args[1:]],
        out_specs=_batch_spec(x.shape, tb),
        out_shape=jax.ShapeDtypeStruct(x.shape, jnp.float32),
    )(*args)


def _down_body(x_ref, lg_ref, lb_ref, w_ref, b_ref, o_ref):
    tb, P, four, C = x_ref.shape
    x = _ln(x_ref[...], lg_ref[...], lb_ref[...], 1e-6)
    Cout = w_ref.shape[-1]
    acc = jnp.zeros((tb * P, Cout), jnp.float32)
    for j in range(4):
        acc = acc + jnp.dot(x[:, :, j, :].reshape(tb * P, C), w_ref[j],
                            preferred_element_type=jnp.float32)
    o_ref[...] = (acc + b_ref[...]).reshape(tb, P, Cout)


def _down_call(xp, lg, lb, w, b):
    B, P, _, C = xp.shape
    Cout = w.shape[-1]
    w4 = w.reshape(4, C, Cout)
    args = (xp, lg.reshape(1, 1, 1, C), lb.reshape(1, 1, 1, C), w4,
            b.reshape(1, Cout))
    TB = B // 2
    return pl.pallas_call(
        _down_body,
        grid=(B // TB,),
        in_specs=[_batch_spec(xp.shape, TB)] + [_full_spec(a.shape) for a in args[1:]],
        out_specs=_batch_spec((B, P, Cout), TB),
        out_shape=jax.ShapeDtypeStruct((B, P, Cout), jnp.float32),
    )(*args)


def _head_body(x_ref, hlg_ref, hlb_ref, hmlg_ref, hmlb_ref, w1_ref, b1_ref,
               w2_ref, b2_ref, w3_ref, b3_ref, halg_ref, halb_ref, wa_ref,
               ba_ref, o_ref):
    f = jnp.mean(x_ref[...], axis=1)
    f = _ln(f, hlg_ref[...], hlb_ref[...], 1e-6)
    h = _gelu(f)
    h = _ln(h, hmlg_ref[...], hmlb_ref[...], 1e-5)
    h = jax.nn.relu(jnp.dot(h, w1_ref[...], preferred_element_type=jnp.float32)
                    + b1_ref[...])
    h = jnp.dot(h, w2_ref[...], preferred_element_type=jnp.float32) + b2_ref[...]
    h = jnp.dot(h, w3_ref[...], preferred_element_type=jnp.float32) + b3_ref[...]
    a = jnp.tanh(jnp.dot(_ln(f, halg_ref[...], halb_ref[...], 1e-5), wa_ref[...],
                         preferred_element_type=jnp.float32) + ba_ref[...])
    o_ref[...] = h * a


def _head_call(x, p, bbp):
    B = x.shape[0]
    r = lambda a: a.reshape(1, -1)
    args = (x, r(bbp['head_lg']), r(bbp['head_lb']), r(p['hm_lg']), r(p['hm_lb']),
            p['hm_w1'], r(p['hm_b1']), p['hm_w2'], r(p['hm_b2']),
            p['hm_w3'], r(p['hm_b3']), r(p['ha_lg']), r(p['ha_lb']),
            p['ha_w'], r(p['ha_b']))
    return pl.pallas_call(
        _head_body,
        grid=(1,),
        in_specs=[_full_spec(a.shape) for a in args],
        out_specs=_full_spec((B, 128)),
        out_shape=jax.ShapeDtypeStruct((B, 128), jnp.float32),
    )(*args)


def _tower_body(x_ref, ws_ref, bs_ref, lag_ref, lab_ref, wa_ref, ba_ref,
                lmg_ref, lmb_ref, wm_ref, bm_ref, wk_ref, bk_ref, o_ref):
    x = x_ref[...]
    h = _gelu(jnp.dot(x, ws_ref[...], preferred_element_type=jnp.float32)
              + bs_ref[...])
    gate = jax.nn.sigmoid(
        jnp.dot(_ln(h, lag_ref[...], lab_ref[...], 1e-5), wa_ref[...],
                preferred_element_type=jnp.float32) + ba_ref[...])
    m = jnp.dot(_ln(h, lmg_ref[...], lmb_ref[...], 1e-5), wm_ref[...],
                preferred_element_type=jnp.float32) + bm_ref[...]
    o_ref[...] = m * gate + (jnp.dot(x, wk_ref[...],
                                     preferred_element_type=jnp.float32)
                             + bk_ref[...])


def _tower_call(x, p):
    B = x.shape[0]
    O = p['wa'].shape[1]
    r = lambda a: a.reshape(1, -1)
    args = (x, p['ws'], r(p['bs']), r(p['lag']), r(p['lab']), p['wa'],
            r(p['ba']), r(p['lmg']), r(p['lmb']), p['wm'], r(p['bm']),
            p['wk'], r(p['bk']))
    return pl.pallas_call(
        _tower_body,
        grid=(1,),
        in_specs=[_full_spec(a.shape) for a in args],
        out_specs=_full_spec((B, O)),
        out_shape=jax.ShapeDtypeStruct((B, O), jnp.float32),
    )(*args)


def _moe_body(f_ref, rw1_ref, rb1_ref, rw2_ref, rb2_ref, ws_ref, bs_ref,
              lag_ref, lab_ref, wa_ref, ba_ref, lmg_ref, lmb_ref, wm_ref,
              bm_ref, wk_ref, bk_ref, o_ref):
    feats = f_ref[...]
    B = feats.shape[0]
    r = jnp.tanh(jnp.dot(feats, rw1_ref[...], preferred_element_type=jnp.float32)
                 + rb1_ref[...])
    fw = jax.nn.sigmoid(jnp.dot(r, rw2_ref[...],
                                preferred_element_type=jnp.float32)
                        + rb2_ref[...])
    col = jax.lax.broadcasted_iota(jnp.int32, fw.shape, 1)
    w0 = jnp.max(fw, axis=1, keepdims=True)
    i0 = jnp.min(jnp.where(fw == w0, col, 127), axis=1, keepdims=True)
    first0 = col == i0
    fw2 = jnp.where(first0, -1.0, fw)
    w1 = jnp.max(fw2, axis=1, keepdims=True)
    i1 = jnp.min(jnp.where(fw2 == w1, col, 127), axis=1, keepdims=True)
    first1 = col == i1
    gates = jnp.where(first0, w0, 0.0) + jnp.where(first1, w1, 0.0)
    moe = jnp.zeros((B, o_ref.shape[1]), jnp.float32)
    for e in range(8):
        h = _gelu(jnp.dot(feats, ws_ref[e], preferred_element_type=jnp.float32)
                  + bs_ref[e])
        g = jax.nn.sigmoid(
            jnp.dot(_ln(h, lag_ref[e], lab_ref[e], 1e-5), wa_ref[e],
                    preferred_element_type=jnp.float32) + ba_ref[e])
        m = jnp.dot(_ln(h, lmg_ref[e], lmb_ref[e], 1e-5), wm_ref[e],
                    preferred_element_type=jnp.float32) + bm_ref[e]
        eo = m * g + (jnp.dot(feats, wk_ref[e],
                              preferred_element_type=jnp.float32) + bk_ref[e])
        moe = moe + gates[:, e:e + 1] * eo
    o_ref[...] = moe


def _moe_call(feats, params):
    B = feats.shape[0]
    ex = params['experts']
    st = lambda k: jnp.stack([e[k] for e in ex])
    st1 = lambda k: jnp.stack([e[k].reshape(1, -1) for e in ex])
    args = (feats, params['r_w1'], params['r_b1'].reshape(1, -1),
            params['r_w2'], params['r_b2'].reshape(1, -1),
            st('ws'), st1('bs'), st1('lag'), st1('lab'), st('wa'), st1('ba'),
            st1('lmg'), st1('lmb'), st('wm'), st1('bm'), st('wk'), st1('bk'))
    return pl.pallas_call(
        _moe_body,
        grid=(1,),
        in_specs=[_full_spec(a.shape) for a in args],
        out_specs=_full_spec((B, 5)),
        out_shape=jax.ShapeDtypeStruct((B, 5), jnp.float32),
    )(*args)


def _patchify2(x):
    B, H, W, C = x.shape
    x = x.reshape(B, H // 2, 2, W // 2, 2, C)
    x = jnp.transpose(x, (0, 1, 3, 2, 4, 5))
    return x.reshape(B, (H // 2) * (W // 2), 4, C)


def _backbone(p, img):
    B = img.shape[0]
    x = jnp.transpose(img, (0, 2, 3, 1))
    x = x.reshape(B, 16, 4, 16, 4, 4)
    x = jnp.transpose(x, (0, 1, 3, 2, 4, 5)).reshape(B, 256, 64)
    x = _stem_call(x, p['stem_w'].reshape(64, 96), p['stem_b'].reshape(1, 96),
                   p['stem_lg'].reshape(1, 96), p['stem_lb'].reshape(1, 96))
    x = x.reshape(B, 16, 16, 96)
    for bp in p['stages'][0]:
        x = _block_hw_call(x, bp, tb=16)
    d = p['downs'][0]
    x = _down_call(_patchify2(x), d['lg'], d['lb'], d['w'], d['b'])
    x = x.reshape(B, 8, 8, 192)
    for bp in p['stages'][1]:
        x = _block_hw_call(x, bp, tb=32)
    d = p['downs'][1]
    x = _down_call(_patchify2(x), d['lg'], d['lb'], d['w'], d['b'])
    for bp in p['stages'][2]:
        x = _block_flat_call(x, bp, hw=4, tb=64)
    d = p['downs'][2]
    x = _down_call(_patchify2(x.reshape(B, 4, 4, 384)), d['lg'], d['lb'],
                   d['w'], d['b'])
    for bp in p['stages'][3]:
        x = _block_flat_call(x, bp, hw=2, tb=B)
    return x


def kernel(metadata, image, params):
    md = metadata
    idx = lambda cols: md[:, jnp.array(cols)]
    x4 = _backbone(params['img']['bb'], image)
    img_f = _head_call(x4, params['img'], params['img']['bb'])
    nsta = _tower_call(idx([0, 2]), params['nst1'])
    nstb = _tower_call(idx([1, 3]), params['nst2'])
    spatial = _tower_call(idx([2, 3, 4]), params['spatial'])
    psf = _tower_call(idx([5, 14]), params['psf'])
    mag = _tower_call(idx([6, 9, 10, 13, 15, 17, 18]), params['mag'])
    coord = _tower_call(idx([7, 8]), params['coord'])
    mega = _tower_call(md[:, :19], params['mega'])
    lc = _tower_call(idx([6, 9, 10, 13, 15, 17, 18, 19, 20, 21, 22, 23]),
                     params['lc'])
    feats = jnp.concatenate([nsta, nstb, spatial, psf, mag, coord, mega,
                             img_f, lc], axis=1)
    return _moe_call(feats, params)


# speedup vs baseline: 2.1160x; 1.6466x over previous
import math

import jax
import jax.numpy as jnp
from jax.experimental import pallas as pl
from jax.experimental.pallas import tpu as pltpu

_SQRT2 = math.sqrt(2.0)


def _gelu(x):
    return 0.5 * x * (1.0 + jax.lax.erf(x / _SQRT2))


def _dot(a, b):
    return jax.lax.dot(a, b, preferred_element_type=jnp.float32)


def _ln(x, g, b, eps):
    m = jnp.mean(x, axis=-1, keepdims=True)
    v = jnp.mean((x - m) ** 2, axis=-1, keepdims=True)
    return (x - m) * jax.lax.rsqrt(v + eps) * g + b


def _full_spec(shape):
    return pl.BlockSpec(shape, lambda i: (0,) * len(shape))


def _bspec(shape, bdim, tb):
    bs = tuple(tb if d == bdim else shape[d] for d in range(len(shape)))
    return pl.BlockSpec(bs, lambda i: tuple(i if d == bdim else 0
                                            for d in range(len(shape))))


def _stem_body(x_ref, w_ref, b_ref, lg_ref, lb_ref, o_ref):
    P, tb, K = x_ref.shape
    C = w_ref.shape[1]
    y = _dot(x_ref[...].reshape(P * tb, K), w_ref[...]) + b_ref[...]
    y = _ln(y, lg_ref[...], lb_ref[...], 1e-6)
    o_ref[...] = y.reshape(16, 16, tb, C)


def _stem_call(xp, w, b, lg, lb):
    P, B, K = xp.shape
    C = w.shape[1]
    TB = 32
    args = (xp, w, b.reshape(1, C), lg.reshape(1, C), lb.reshape(1, C))
    return pl.pallas_call(
        _stem_body,
        grid=(B // TB,),
        in_specs=[_bspec(xp.shape, 1, TB)] + [_full_spec(a.shape) for a in args[1:]],
        out_specs=_bspec((16, 16, B, C), 2, TB),
        out_shape=jax.ShapeDtypeStruct((16, 16, B, C), jnp.float32),
    )(*args)


def _block_body(x_ref, dwk_ref, dwb_ref, lg_ref, lb_ref, w1_ref, b1_ref,
                w2_ref, b2_ref, g_ref, o_ref, pad_ref, *, patch_out):
    H, W, tb, C = x_ref.shape
    x = x_ref[...]
    pad_ref[...] = jnp.zeros(pad_ref.shape, jnp.float32)
    pad_ref[3:H + 3, 3:W + 3, :, :] = x
    y = jnp.zeros((H, W, tb, C), jnp.float32)
    for k in range(49):
        dh, dw = k // 7, k % 7
        y = y + pad_ref[dh:dh + H, dw:dw + W, :, :] * dwk_ref[k]
    y = y + dwb_ref[...]
    y = _ln(y, lg_ref[...], lb_ref[...], 1e-6)
    h = _dot(y.reshape(H * W * tb, C), w1_ref[...]) + b1_ref[...]
    h = _gelu(h)
    o = _dot(h, w2_ref[...]) + b2_ref[...]
    o = x + g_ref[...] * o.reshape(H, W, tb, C)
    if patch_out:
        op = o.reshape(H // 2, 2, W // 2, 2, tb, C)
        for j in range(4):
            dh, dw = j // 2, j % 2
            o_ref[:, :, j, :, :] = op[:, dh, :, dw, :, :]
    else:
        o_ref[...] = o


def _block_call(x, p, tb, patch_out=False):
    H, W, B, C = x.shape
    dwk = p['dw'].reshape(49, C)
    args = (x, dwk, p['dwb'].reshape(1, C), p['lg'].reshape(1, C),
            p['lb'].reshape(1, C), p['w1'], p['b1'].reshape(1, -1),
            p['w2'], p['b2'].reshape(1, C), p['g'].reshape(1, C))
    if patch_out:
        oshape = (H // 2, W // 2, 4, B, C)
        ospec = _bspec(oshape, 3, tb)
    else:
        oshape = (H, W, B, C)
        ospec = _bspec(oshape, 2, tb)
    body = lambda *refs: _block_body(*refs, patch_out=patch_out)
    return pl.pallas_call(
        body,
        grid=(B // tb,),
        in_specs=[_bspec(x.shape, 2, tb)] + [_full_spec(a.shape) for a in args[1:]],
        out_specs=ospec,
        out_shape=jax.ShapeDtypeStruct(oshape, jnp.float32),
        scratch_shapes=[pltpu.VMEM((H + 6, W + 6, tb, C), jnp.float32)],
    )(*args)


def _down_body(x_ref, lg_ref, lb_ref, w_ref, b_ref, o_ref):
    Ph, Pw, _, tb, C = x_ref.shape
    Cout = w_ref.shape[-1]
    xl = _ln(x_ref[...], lg_ref[...], lb_ref[...], 1e-6)
    acc = jnp.zeros((Ph * Pw * tb, Cout), jnp.float32)
    for j in range(4):
        acc = acc + _dot(xl[:, :, j, :, :].reshape(Ph * Pw * tb, C), w_ref[j])
    o_ref[...] = (acc + b_ref[...]).reshape(Ph, Pw, tb, Cout)


def _down_call(xp, d):
    Ph, Pw, _, B, C = xp.shape
    Cout = d['w'].shape[-1]
    args = (xp, d['lg'].reshape(1, C), d['lb'].reshape(1, C),
            d['w'].reshape(4, C, Cout), d['b'].reshape(1, Cout))
    TB = B // 2
    return pl.pallas_call(
        _down_body,
        grid=(B // TB,),
        in_specs=[_bspec(xp.shape, 3, TB)] + [_full_spec(a.shape) for a in args[1:]],
        out_specs=_bspec((Ph, Pw, B, Cout), 2, TB),
        out_shape=jax.ShapeDtypeStruct((Ph, Pw, B, Cout), jnp.float32),
    )(*args)


def _head_body(x_ref, hlg_ref, hlb_ref, hmlg_ref, hmlb_ref, w1_ref, b1_ref,
               w2_ref, b2_ref, w3_ref, b3_ref, halg_ref, halb_ref, wa_ref,
               ba_ref, o_ref):
    f = jnp.mean(x_ref[...], axis=(0, 1))
    f = _ln(f, hlg_ref[...], hlb_ref[...], 1e-6)
    h = _gelu(f)
    h = _ln(h, hmlg_ref[...], hmlb_ref[...], 1e-5)
    h = jax.nn.relu(_dot(h, w1_ref[...]) + b1_ref[...])
    h = _dot(h, w2_ref[...]) + b2_ref[...]
    h = _dot(h, w3_ref[...]) + b3_ref[...]
    a = jnp.tanh(_dot(_ln(f, halg_ref[...], halb_ref[...], 1e-5),
                      wa_ref[...]) + ba_ref[...])
    o_ref[...] = h * a


def _head_call(x, p, bbp):
    B = x.shape[2]
    r = lambda a: a.reshape(1, -1)
    args = (x, r(bbp['head_lg']), r(bbp['head_lb']), r(p['hm_lg']),
            r(p['hm_lb']), p['hm_w1'], r(p['hm_b1']), p['hm_w2'],
            r(p['hm_b2']), p['hm_w3'], r(p['hm_b3']), r(p['ha_lg']),
            r(p['ha_lb']), p['ha_w'], r(p['ha_b']))
    return pl.pallas_call(
        _head_body,
        grid=(1,),
        in_specs=[_full_spec(a.shape) for a in args],
        out_specs=_full_spec((B, 128)),
        out_shape=jax.ShapeDtypeStruct((B, 128), jnp.float32),
    )(*args)


def _tower_body(x_ref, ws_ref, bs_ref, lag_ref, lab_ref, wa_ref, ba_ref,
                lmg_ref, lmb_ref, wm_ref, bm_ref, wk_ref, bk_ref, o_ref):
    x = x_ref[...]
    h = _gelu(_dot(x, ws_ref[...]) + bs_ref[...])
    gate = jax.nn.sigmoid(
        _dot(_ln(h, lag_ref[...], lab_ref[...], 1e-5), wa_ref[...])
        + ba_ref[...])
    m = _dot(_ln(h, lmg_ref[...], lmb_ref[...], 1e-5), wm_ref[...]) + bm_ref[...]
    o_ref[...] = m * gate + (_dot(x, wk_ref[...]) + bk_ref[...])


def _tower_call(x, p):
    B = x.shape[0]
    O = p['wa'].shape[1]
    r = lambda a: a.reshape(1, -1)
    args = (x, p['ws'], r(p['bs']), r(p['lag']), r(p['lab']), p['wa'],
            r(p['ba']), r(p['lmg']), r(p['lmb']), p['wm'], r(p['bm']),
            p['wk'], r(p['bk']))
    return pl.pallas_call(
        _tower_body,
        grid=(1,),
        in_specs=[_full_spec(a.shape) for a in args],
        out_specs=_full_spec((B, O)),
        out_shape=jax.ShapeDtypeStruct((B, O), jnp.float32),
    )(*args)


def _moe_body(f_ref, rw1_ref, rb1_ref, rw2_ref, rb2_ref, ws_ref, bs_ref,
              lag_ref, lab_ref, wa_ref, ba_ref, lmg_ref, lmb_ref, wm_ref,
              bm_ref, wk_ref, bk_ref, o_ref):
    feats = f_ref[...]
    B = feats.shape[0]
    r = jnp.tanh(_dot(feats, rw1_ref[...]) + rb1_ref[...])
    fw = jax.nn.sigmoid(_dot(r, rw2_ref[...]) + rb2_ref[...])
    col = jax.lax.broadcasted_iota(jnp.int32, fw.shape, 1)
    w0 = jnp.max(fw, axis=1, keepdims=True)
    i0 = jnp.min(jnp.where(fw == w0, col, 127), axis=1, keepdims=True)
    first0 = col == i0
    fw2 = jnp.where(first0, -1.0, fw)
    w1 = jnp.max(fw2, axis=1, keepdims=True)
    i1 = jnp.min(jnp.where(fw2 == w1, col, 127), axis=1, keepdims=True)
    first1 = col == i1
    gates = jnp.where(first0, w0, 0.0) + jnp.where(first1, w1, 0.0)
    moe = jnp.zeros((B, o_ref.shape[1]), jnp.float32)
    for e in range(8):
        h = _gelu(_dot(feats, ws_ref[e]) + bs_ref[e])
        g = jax.nn.sigmoid(
            _dot(_ln(h, lag_ref[e], lab_ref[e], 1e-5), wa_ref[e]) + ba_ref[e])
        m = _dot(_ln(h, lmg_ref[e], lmb_ref[e], 1e-5), wm_ref[e]) + bm_ref[e]
        eo = m * g + (_dot(feats, wk_ref[e]) + bk_ref[e])
        moe = moe + gates[:, e:e + 1] * eo
    o_ref[...] = moe


def _moe_call(feats, params):
    B = feats.shape[0]
    ex = params['experts']
    st = lambda k: jnp.stack([e[k] for e in ex])
    st1 = lambda k: jnp.stack([e[k].reshape(1, -1) for e in ex])
    args = (feats, params['r_w1'], params['r_b1'].reshape(1, -1),
            params['r_w2'], params['r_b2'].reshape(1, -1),
            st('ws'), st1('bs'), st1('lag'), st1('lab'), st('wa'), st1('ba'),
            st1('lmg'), st1('lmb'), st('wm'), st1('bm'), st('wk'), st1('bk'))
    return pl.pallas_call(
        _moe_body,
        grid=(1,),
        in_specs=[_full_spec(a.shape) for a in args],
        out_specs=_full_spec((B, 5)),
        out_shape=jax.ShapeDtypeStruct((B, 5), jnp.float32),
    )(*args)


def _backbone(p, img):
    B = img.shape[0]
    x = img.reshape(B, 4, 16, 4, 16, 4)
    x = jnp.transpose(x, (2, 4, 0, 3, 5, 1))
    x = x.reshape(256, B, 64)
    x = _stem_call(x, p['stem_w'].reshape(64, 96), p['stem_b'],
                   p['stem_lg'], p['stem_lb'])
    tbs = [16, 32, 64, 64]
    for si in range(4):
        if si > 0:
            x = _down_call(x, p['downs'][si - 1])
        blocks = p['stages'][si]
        for bi, bp in enumerate(blocks):
            last = bi == len(blocks) - 1 and si < 3
            x = _block_call(x, bp, tb=tbs[si], patch_out=last)
    return x


def kernel(metadata, image, params):
    md = metadata
    idx = lambda cols: md[:, jnp.array(cols)]
    x4 = _backbone(params['img']['bb'], image)
    img_f = _head_call(x4, params['img'], params['img']['bb'])
    nsta = _tower_call(idx([0, 2]), params['nst1'])
    nstb = _tower_call(idx([1, 3]), params['nst2'])
    spatial = _tower_call(idx([2, 3, 4]), params['spatial'])
    psf = _tower_call(idx([5, 14]), params['psf'])
    mag = _tower_call(idx([6, 9, 10, 13, 15, 17, 18]), params['mag'])
    coord = _tower_call(idx([7, 8]), params['coord'])
    mega = _tower_call(md[:, :19], params['mega'])
    lc = _tower_call(idx([6, 9, 10, 13, 15, 17, 18, 19, 20, 21, 22, 23]),
                     params['lc'])
    feats = jnp.concatenate([nsta, nstb, spatial, psf, mag, coord, mega,
                             img_f, lc], axis=1)
    return _moe_call(feats, params)


# speedup vs baseline: 2.1623x; 1.0219x over previous
import math

import jax
import jax.numpy as jnp
from jax.experimental import pallas as pl
from jax.experimental.pallas import tpu as pltpu

_SQRT2 = math.sqrt(2.0)


def _gelu(x):
    return 0.5 * x * (1.0 + jax.lax.erf(x / _SQRT2))


def _dot(a, b):
    return jax.lax.dot(a, b, preferred_element_type=jnp.float32)


def _ln(x, g, b, eps):
    m = jnp.mean(x, axis=-1, keepdims=True)
    v = jnp.mean((x - m) ** 2, axis=-1, keepdims=True)
    return (x - m) * jax.lax.rsqrt(v + eps) * g + b


def _full_spec(shape):
    return pl.BlockSpec(shape, lambda i: (0,) * len(shape))


def _bspec(shape, bdim, tb):
    bs = tuple(tb if d == bdim else shape[d] for d in range(len(shape)))
    return pl.BlockSpec(bs, lambda i: tuple(i if d == bdim else 0
                                            for d in range(len(shape))))


_BLK_KEYS = ('dw', 'dwb', 'lg', 'lb', 'w1', 'b1', 'w2', 'b2', 'g')


def _blk_args(bp):
    C = bp['dwb'].shape[-1]
    return (bp['dw'].reshape(49, C), bp['dwb'].reshape(1, C),
            bp['lg'].reshape(1, C), bp['lb'].reshape(1, C), bp['w1'],
            bp['b1'].reshape(1, -1), bp['w2'], bp['b2'].reshape(1, C),
            bp['g'].reshape(1, C))


def _blk_tail(x, y, dwb, lg, lb, w1, b1, w2, b2, g):
    H, W, tb, C = x.shape
    y = y + dwb
    y = _ln(y, lg, lb, 1e-6)
    h = _gelu(_dot(y.reshape(H * W * tb, C), w1) + b1)
    o = _dot(h, w2) + b2
    return x + g * o.reshape(H, W, tb, C)


def _blk_taps(x, pad_ref, dwk, dwb, lg, lb, w1, b1, w2, b2, g):
    H, W, tb, C = x.shape
    pad_ref[...] = jnp.zeros(pad_ref.shape, jnp.float32)
    pad_ref[3:H + 3, 3:W + 3, :, :] = x
    y = jnp.zeros((H, W, tb, C), jnp.float32)
    for k in range(49):
        dh, dw = divmod(k, 7)
        y = y + pad_ref[dh:dh + H, dw:dw + W, :, :] * dwk[k]
    return _blk_tail(x, y, dwb, lg, lb, w1, b1, w2, b2, g)


def _blk_pairs(x, kq, dwb, lg, lb, w1, b1, w2, b2, g):
    H, W, tb, C = x.shape
    y = jnp.zeros((H, W, tb, C), jnp.float32)
    for q in range(H * W):
        qh, qw = divmod(q, W)
        y = y + x[qh, qw][None, None] * kq[q]
    return _blk_tail(x, y, dwb, lg, lb, w1, b1, w2, b2, g)


def _kq(bp, H, W):
    C = bp['dwb'].shape[-1]
    k2 = bp['dw'].reshape(7, 7, C)
    rows = []
    for q in range(H * W):
        qh, qw = divmod(q, W)
        row = jnp.stack([jnp.stack([k2[qh - ph + 3, qw - pw + 3]
                                    for pw in range(W)]) for ph in range(H)])
        rows.append(row[:, :, None, :])
    return jnp.stack(rows)


def _down_v(x, lg, lb, w4, b):
    H, W, tb, C = x.shape
    Cout = w4.shape[-1]
    xl = _ln(x, lg, lb, 1e-6)
    xp = xl.reshape(H // 2, 2, W // 2, 2, tb, C)
    acc = jnp.zeros((H // 2 * (W // 2) * tb, Cout), jnp.float32)
    for j in range(4):
        dh, dw = divmod(j, 2)
        acc = acc + _dot(xp[:, dh, :, dw].reshape(-1, C), w4[j])
    return (acc + b).reshape(H // 2, W // 2, tb, Cout)


def _down_args(d):
    C, Cout = d['w'].shape[2], d['w'].shape[3]
    return (d['lg'].reshape(1, C), d['lb'].reshape(1, C),
            d['w'].reshape(4, C, Cout), d['b'].reshape(1, Cout))


def _stage1_call(xp, p, B):
    TB = 16
    blocks = p['stages'][0]
    args = [xp, p['stem_w'].reshape(64, 96), p['stem_b'].reshape(1, 96),
            p['stem_lg'].reshape(1, 96), p['stem_lb'].reshape(1, 96)]
    for bp in blocks:
        args.extend(_blk_args(bp))
    args.extend(_down_args(p['downs'][0]))

    def body(*refs):
        xp_ref, sw, sb, slg, slb = refs[:5]
        o_ref, pad_ref = refs[-2], refs[-1]
        P, tb, K = xp_ref.shape
        y = _dot(xp_ref[...].reshape(P * tb, K), sw[...]) + sb[...]
        y = _ln(y, slg[...], slb[...], 1e-6)
        x = y.reshape(16, 16, tb, 96)
        for bi in range(3):
            br = refs[5 + 9 * bi:5 + 9 * (bi + 1)]
            x = _blk_taps(x, pad_ref, *[r[...] for r in br])
        dr = refs[5 + 27:5 + 31]
        o_ref[...] = _down_v(x, *[r[...] for r in dr])

    return pl.pallas_call(
        body,
        grid=(B // TB,),
        in_specs=[_bspec(xp.shape, 1, TB)] + [_full_spec(a.shape)
                                              for a in args[1:]],
        out_specs=_bspec((8, 8, B, 192), 2, TB),
        out_shape=jax.ShapeDtypeStruct((8, 8, B, 192), jnp.float32),
        scratch_shapes=[pltpu.VMEM((22, 22, TB, 96), jnp.float32)],
    )(*args)


def _stage2_call(x, p, B):
    TB = 32
    blocks = p['stages'][1]
    args = [x]
    for bp in blocks:
        args.extend(_blk_args(bp))
    args.extend(_down_args(p['downs'][1]))

    def body(*refs):
        x_ref = refs[0]
        o_ref, pad_ref = refs[-2], refs[-1]
        x = x_ref[...]
        for bi in range(3):
            br = refs[1 + 9 * bi:1 + 9 * (bi + 1)]
            x = _blk_taps(x, pad_ref, *[r[...] for r in br])
        dr = refs[1 + 27:1 + 31]
        o_ref[...] = _down_v(x, *[r[...] for r in dr])

    return pl.pallas_call(
        body,
        grid=(B // TB,),
        in_specs=[_bspec(x.shape, 2, TB)] + [_full_spec(a.shape)
                                             for a in args[1:]],
        out_specs=_bspec((4, 4, B, 384), 2, TB),
        out_shape=jax.ShapeDtypeStruct((4, 4, B, 384), jnp.float32),
        scratch_shapes=[pltpu.VMEM((14, 14, TB, 192), jnp.float32)],
    )(*args)


def _stage3_call(x, p, B):
    TB = 64
    blocks = p['stages'][2]

    def run(x, blks, down):
        C = 384
        args = [x]
        for bp in blks:
            a = list(_blk_args(bp))
            a[0] = _kq(bp, 4, 4)
            args.extend(a)
        if down is not None:
            args.extend(_down_args(down))
            oshape = (2, 2, B, 768)
        else:
            oshape = (4, 4, B, C)
        nb = len(blks)

        def body(*refs):
            x_ref, o_ref = refs[0], refs[-1]
            x = x_ref[...]
            for bi in range(nb):
                br = refs[1 + 9 * bi:1 + 9 * (bi + 1)]
                x = _blk_pairs(x, *[r[...] for r in br])
            if down is not None:
                dr = refs[1 + 9 * nb:1 + 9 * nb + 4]
                x = _down_v(x, *[r[...] for r in dr])
            o_ref[...] = x

        return pl.pallas_call(
            body,
            grid=(B // TB,),
            in_specs=[_bspec(x.shape, 2, TB)] + [_full_spec(a.shape)
                                                 for a in args[1:]],
            out_specs=_bspec(oshape, 2, TB),
            out_shape=jax.ShapeDtypeStruct(oshape, jnp.float32),
        )(*args)

    x = run(x, blocks[:5], None)
    return run(x, blocks[5:], p['downs'][2])


def _stage4_call(x, p, B):
    TB = 64

    def body(*refs):
        x_ref, o_ref = refs[0], refs[-1]
        o_ref[...] = _blk_pairs(x_ref[...], *[r[...] for r in refs[1:-1]])

    for bp in p['stages'][3]:
        a = list(_blk_args(bp))
        a[0] = _kq(bp, 2, 2)
        args = [x] + a
        x = pl.pallas_call(
            body,
            grid=(B // TB,),
            in_specs=[_bspec(x.shape, 2, TB)] + [_full_spec(w.shape)
                                                 for w in args[1:]],
            out_specs=_bspec((2, 2, B, 768), 2, TB),
            out_shape=jax.ShapeDtypeStruct((2, 2, B, 768), jnp.float32),
        )(*args)
    return x


def _head_body(x_ref, hlg_ref, hlb_ref, hmlg_ref, hmlb_ref, w1_ref, b1_ref,
               w2_ref, b2_ref, w3_ref, b3_ref, halg_ref, halb_ref, wa_ref,
               ba_ref, o_ref):
    f = jnp.mean(x_ref[...], axis=(0, 1))
    f = _ln(f, hlg_ref[...], hlb_ref[...], 1e-6)
    h = _gelu(f)
    h = _ln(h, hmlg_ref[...], hmlb_ref[...], 1e-5)
    h = jax.nn.relu(_dot(h, w1_ref[...]) + b1_ref[...])
    h = _dot(h, w2_ref[...]) + b2_ref[...]
    h = _dot(h, w3_ref[...]) + b3_ref[...]
    a = jnp.tanh(_dot(_ln(f, halg_ref[...], halb_ref[...], 1e-5),
                      wa_ref[...]) + ba_ref[...])
    o_ref[...] = h * a


def _head_call(x, p, bbp):
    B = x.shape[2]
    r = lambda a: a.reshape(1, -1)
    args = (x, r(bbp['head_lg']), r(bbp['head_lb']), r(p['hm_lg']),
            r(p['hm_lb']), p['hm_w1'], r(p['hm_b1']), p['hm_w2'],
            r(p['hm_b2']), p['hm_w3'], r(p['hm_b3']), r(p['ha_lg']),
            r(p['ha_lb']), p['ha_w'], r(p['ha_b']))
    return pl.pallas_call(
        _head_body,
        grid=(1,),
        in_specs=[_full_spec(a.shape) for a in args],
        out_specs=_full_spec((B, 128)),
        out_shape=jax.ShapeDtypeStruct((B, 128), jnp.float32),
    )(*args)


_TOWER_KEYS = ('ws', 'bs', 'lag', 'lab', 'wa', 'ba', 'lmg', 'lmb', 'wm',
               'bm', 'wk', 'bk')


def _tower_args(p):
    r = lambda a: a.reshape(1, -1)
    return (p['ws'], r(p['bs']), r(p['lag']), r(p['lab']), p['wa'],
            r(p['ba']), r(p['lmg']), r(p['lmb']), p['wm'], r(p['bm']),
            p['wk'], r(p['bk']))


def _tower_v(x, ws, bs, lag, lab, wa, ba, lmg, lmb, wm, bm, wk, bk):
    h = _gelu(_dot(x, ws) + bs)
    gate = jax.nn.sigmoid(_dot(_ln(h, lag, lab, 1e-5), wa) + ba)
    m = _dot(_ln(h, lmg, lmb, 1e-5), wm) + bm
    return m * gate + (_dot(x, wk) + bk)


def _tower_body(x_ref, *refs):
    o_ref = refs[-1]
    o_ref[...] = _tower_v(x_ref[...], *[r[...] for r in refs[:-1]])


def _tower_call(x, p):
    B = x.shape[0]
    O = p['wa'].shape[1]
    args = (x,) + _tower_args(p)
    return pl.pallas_call(
        _tower_body,
        grid=(1,),
        in_specs=[_full_spec(a.shape) for a in args],
        out_specs=_full_spec((B, O)),
        out_shape=jax.ShapeDtypeStruct((B, O), jnp.float32),
    )(*args)


def _moe_body(*refs):
    f_ref, rw1_ref, rb1_ref, rw2_ref, rb2_ref = refs[:5]
    o_ref = refs[-1]
    feats = f_ref[...]
    B = feats.shape[0]
    r = jnp.tanh(_dot(feats, rw1_ref[...]) + rb1_ref[...])
    fw = jax.nn.sigmoid(_dot(r, rw2_ref[...]) + rb2_ref[...])
    col = jax.lax.broadcasted_iota(jnp.int32, fw.shape, 1)
    w0 = jnp.max(fw, axis=1, keepdims=True)
    i0 = jnp.min(jnp.where(fw == w0, col, 127), axis=1, keepdims=True)
    first0 = col == i0
    fw2 = jnp.where(first0, -1.0, fw)
    w1 = jnp.max(fw2, axis=1, keepdims=True)
    i1 = jnp.min(jnp.where(fw2 == w1, col, 127), axis=1, keepdims=True)
    first1 = col == i1
    gates = jnp.where(first0, w0, 0.0) + jnp.where(first1, w1, 0.0)
    moe = jnp.zeros((B, o_ref.shape[1]), jnp.float32)
    for e in range(8):
        er = refs[5 + 12 * e:5 + 12 * (e + 1)]
        eo = _tower_v(feats, *[r_[...] for r_ in er])
        moe = moe + gates[:, e:e + 1] * eo
    o_ref[...] = moe


def _moe_call(feats, params):
    B = feats.shape[0]
    args = [feats, params['r_w1'], params['r_b1'].reshape(1, -1),
            params['r_w2'], params['r_b2'].reshape(1, -1)]
    for e in params['experts']:
        args.extend(_tower_args(e))
    return pl.pallas_call(
        _moe_body,
        grid=(1,),
        in_specs=[_full_spec(a.shape) for a in args],
        out_specs=_full_spec((B, 5)),
        out_shape=jax.ShapeDtypeStruct((B, 5), jnp.float32),
    )(*args)


def kernel(metadata, image, params):
    md = metadata
    idx = lambda cols: md[:, jnp.array(cols)]
    B = image.shape[0]
    bb = params['img']['bb']
    xp = image.reshape(B, 4, 16, 4, 16, 4)
    xp = jnp.transpose(xp, (2, 4, 0, 3, 5, 1)).reshape(256, B, 64)
    x = _stage1_call(xp, bb, B)
    x = _stage2_call(x, bb, B)
    x = _stage3_call(x, bb, B)
    x4 = _stage4_call(x, bb, B)
    img_f = _head_call(x4, params['img'], bb)
    nsta = _tower_call(idx([0, 2]), params['nst1'])
    nstb = _tower_call(idx([1, 3]), params['nst2'])
    spatial = _tower_call(idx([2, 3, 4]), params['spatial'])
    psf = _tower_call(idx([5, 14]), params['psf'])
    mag = _tower_call(idx([6, 9, 10, 13, 15, 17, 18]), params['mag'])
    coord = _tower_call(idx([7, 8]), params['coord'])
    mega = _tower_call(md[:, :19], params['mega'])
    lc = _tower_call(idx([6, 9, 10, 13, 15, 17, 18, 19, 20, 21, 22, 23]),
                     params['lc'])
    feats = jnp.concatenate([nsta, nstb, spatial, psf, mag, coord, mega,
                             img_f, lc], axis=1)
    return _moe_call(feats, params)
